# Initial kernel scaffold; baseline (speedup 1.0000x reference)
import jax
import jax.numpy as jnp
from jax.experimental import pallas as pl


def kernel(x, edge_index, W1, b1, Wmu, bmu, Wls, bls):
    raise NotImplementedError("write your pallas kernel here")


# speedup vs baseline: 20.6477x; 20.6477x over previous
import functools

import jax
import jax.numpy as jnp
from jax import lax
from jax.experimental import pallas as pl
from jax.experimental.pallas import tpu as pltpu
from jax.experimental.pallas import tpu_sc as plsc

N = 10000
E = 320000
D_IN = 128
D_H = 64
D_OUT = 32

NC = 2
NS = 16
W = 128
KW = 80
EP = NC * NS * KW * W
NP = 10240
RPT = NP // NS

_MESH = plsc.VectorSubcoreMesh(core_axis_name="c", subcore_axis_name="s")
_SC_PARAMS = pltpu.CompilerParams(use_tc_tiling_on_sc=False)


def _sc_degree(dstw, ones1, zeros1):
    @functools.partial(
        pl.kernel,
        out_type=jax.ShapeDtypeStruct((NC, NP, 16), jnp.float32),
        mesh=_MESH,
        compiler_params=_SC_PARAMS,
        scratch_types=[
            pltpu.VMEM((KW, W), jnp.int32),
            pltpu.VMEM((W, 16), jnp.float32),
            pltpu.VMEM_SHARED((NP, 16), jnp.float32),
        ],
    )
    def k(dst_hbm, ones_hbm, z_hbm, out_hbm, dst_v, ones_v, acc):
        cid = lax.axis_index("c")
        sid = lax.axis_index("s")
        base = sid * RPT
        pltpu.sync_copy(z_hbm.at[pl.ds(base, RPT)], acc.at[pl.ds(base, RPT)])
        pltpu.sync_copy(ones_hbm, ones_v)
        pltpu.sync_copy(dst_hbm.at[cid].at[sid], dst_v)
        plsc.subcore_barrier()

        @pl.loop(0, KW)
        def _(t):
            pltpu.sync_copy(ones_v, acc.at[dst_v.at[t]], add=True)

        plsc.subcore_barrier()
        pltpu.sync_copy(acc.at[pl.ds(base, RPT)],
                        out_hbm.at[cid].at[pl.ds(base, RPT)])

    return k(dstw, ones1, zeros1)


def _sc_segment_sum(y, srcw, dstw, zeros64):
    @functools.partial(
        pl.kernel,
        out_type=jax.ShapeDtypeStruct((NC, NP, D_H), jnp.float32),
        mesh=_MESH,
        compiler_params=_SC_PARAMS,
        scratch_types=[
            pltpu.VMEM((KW, W), jnp.int32),
            pltpu.VMEM((KW, W), jnp.int32),
            pltpu.VMEM((W, D_H), jnp.float32),
            pltpu.VMEM((W, D_H), jnp.float32),
            pltpu.VMEM_SHARED((NP, D_H), jnp.float32),
            pltpu.SemaphoreType.DMA,
            pltpu.SemaphoreType.DMA,
        ],
    )
    def k(y_hbm, src_hbm, dst_hbm, z_hbm, out_hbm,
          src_v, dst_v, rows0, rows1, acc, sem0, sem1):
        cid = lax.axis_index("c")
        sid = lax.axis_index("s")
        base = sid * RPT
        pltpu.sync_copy(z_hbm.at[pl.ds(base, RPT)], acc.at[pl.ds(base, RPT)])
        pltpu.sync_copy(src_hbm.at[cid].at[sid], src_v)
        pltpu.sync_copy(dst_hbm.at[cid].at[sid], dst_v)
        plsc.subcore_barrier()

        pltpu.async_copy(y_hbm.at[src_v.at[0]], rows0, sem0)

        @pl.loop(0, KW, step=2)
        def _(t):
            pltpu.make_async_copy(y_hbm.at[src_v.at[t]], rows0, sem0).wait()
            pltpu.async_copy(y_hbm.at[src_v.at[t + 1]], rows1, sem1)
            pltpu.sync_copy(rows0, acc.at[dst_v.at[t]], add=True)
            pltpu.make_async_copy(y_hbm.at[src_v.at[t + 1]], rows1, sem1).wait()

            @pl.when(t + 2 < KW)
            def _():
                pltpu.async_copy(y_hbm.at[src_v.at[t + 2]], rows0, sem0)

            pltpu.sync_copy(rows1, acc.at[dst_v.at[t + 1]], add=True)

        plsc.subcore_barrier()
        pltpu.sync_copy(acc.at[pl.ds(base, RPT)],
                        out_hbm.at[cid].at[pl.ds(base, RPT)])

    return k(y, srcw, dstw, zeros64)


def _tc_layer1(cnt, x, W1):
    def body(c_ref, x_ref, w_ref, y_ref, d_ref):
        c = c_ref[0][:, 0:1] + c_ref[1][:, 0:1]
        d = lax.rsqrt(c + 1.0)[:N]
        u = jax.lax.dot_general(
            x_ref[...], w_ref[...], (((1,), (0,)), ((), ())),
            preferred_element_type=jnp.float32,
            precision=lax.Precision.HIGHEST)
        y_ref[...] = u * d
        d_ref[...] = d

    return pl.pallas_call(
        body,
        out_shape=(jax.ShapeDtypeStruct((N, D_H), jnp.float32),
                   jax.ShapeDtypeStruct((N, 1), jnp.float32)),
    )(cnt, x, W1)


def _tc_layer2(parts1, y1, d, Wcat, b1):
    def body(p_ref, y1_ref, d_ref, w_ref, b_ref, y2_ref):
        s1 = p_ref[0][:N] + p_ref[1][:N]
        dv = d_ref[...]
        h = jnp.maximum(dv * (s1 + y1_ref[...]) + b_ref[...], 0.0)
        y2_ref[...] = jax.lax.dot_general(
            h, w_ref[...], (((1,), (0,)), ((), ())),
            preferred_element_type=jnp.float32,
            precision=lax.Precision.HIGHEST) * dv

    return pl.pallas_call(
        body,
        out_shape=jax.ShapeDtypeStruct((N, D_H), jnp.float32),
    )(parts1, y1, d, Wcat, b1)


def _tc_out(parts2, y2, d, bcat):
    def body(p_ref, y2_ref, d_ref, b_ref, mu_ref, ls_ref):
        s2 = p_ref[0][:N] + p_ref[1][:N]
        o = d_ref[...] * (s2 + y2_ref[...]) + b_ref[...]
        mu_ref[...] = o[:, :D_OUT]
        ls_ref[...] = o[:, D_OUT:]

    return pl.pallas_call(
        body,
        out_shape=(jax.ShapeDtypeStruct((N, D_OUT), jnp.float32),
                   jax.ShapeDtypeStruct((N, D_OUT), jnp.float32)),
    )(parts2, y2, d, bcat)


def kernel(x, edge_index, W1, b1, Wmu, bmu, Wls, bls):
    src = edge_index[0]
    dst = edge_index[1]
    pad = EP - E
    srcw = jnp.concatenate(
        [src, jnp.zeros((pad,), jnp.int32)]).reshape(NC, NS, KW, W)
    dstw = jnp.concatenate(
        [dst, jnp.full((pad,), N, jnp.int32)]).reshape(NC, NS, KW, W)
    zeros1 = jnp.zeros((NP, 16), jnp.float32)
    zeros64 = jnp.zeros((NP, D_H), jnp.float32)
    ones1 = jnp.zeros((W, 16), jnp.float32).at[:, 0].set(1.0)

    cnt = _sc_degree(dstw, ones1, zeros1)
    y1, d = _tc_layer1(cnt, x, W1)
    parts1 = _sc_segment_sum(y1, srcw, dstw, zeros64)
    Wcat = jnp.concatenate([Wmu, Wls], axis=1)
    bcat = jnp.concatenate([bmu, bls]).reshape(1, 2 * D_OUT)
    y2 = _tc_layer2(parts1, y1, d, Wcat, b1.reshape(1, D_H))
    parts2 = _sc_segment_sum(y2, srcw, dstw, zeros64)
    mu, logstd = _tc_out(parts2, y2, d, bcat)
    return (mu, logstd)


# speedup vs baseline: 20.6949x; 1.0023x over previous
import functools

import jax
import jax.numpy as jnp
from jax import lax
from jax.experimental import pallas as pl
from jax.experimental.pallas import tpu as pltpu
from jax.experimental.pallas import tpu_sc as plsc

N = 10000
E = 320000
D_IN = 128
D_H = 64
D_OUT = 32

NC = 2
NS = 16
W = 128
KW = 80
EP = NC * NS * KW * W
NP = 10240
RPT = NP // NS

_MESH = plsc.VectorSubcoreMesh(core_axis_name="c", subcore_axis_name="s")
_SC_PARAMS = pltpu.CompilerParams(use_tc_tiling_on_sc=False)


def _sc_degree(dstw, ones1, zeros1):
    @functools.partial(
        pl.kernel,
        out_type=jax.ShapeDtypeStruct((NC, NP, 16), jnp.float32),
        mesh=_MESH,
        compiler_params=_SC_PARAMS,
        scratch_types=[
            pltpu.VMEM((KW, W), jnp.int32),
            pltpu.VMEM((W, 16), jnp.float32),
            pltpu.VMEM_SHARED((NP, 16), jnp.float32),
        ],
    )
    def k(dst_hbm, ones_hbm, z_hbm, out_hbm, dst_v, ones_v, acc):
        cid = lax.axis_index("c")
        sid = lax.axis_index("s")
        base = sid * RPT
        pltpu.sync_copy(z_hbm.at[pl.ds(base, RPT)], acc.at[pl.ds(base, RPT)])
        pltpu.sync_copy(ones_hbm, ones_v)
        pltpu.sync_copy(dst_hbm.at[cid].at[sid], dst_v)
        plsc.subcore_barrier()

        @pl.loop(0, KW)
        def _(t):
            pltpu.sync_copy(ones_v, acc.at[dst_v.at[t]], add=True)

        plsc.subcore_barrier()
        pltpu.sync_copy(acc.at[pl.ds(base, RPT)],
                        out_hbm.at[cid].at[pl.ds(base, RPT)])

    return k(dstw, ones1, zeros1)


def _sc_segment_sum(y, srcw, dstw, zeros64):
    @functools.partial(
        pl.kernel,
        out_type=jax.ShapeDtypeStruct((NC, NP, D_H), jnp.float32),
        mesh=_MESH,
        compiler_params=_SC_PARAMS,
        scratch_types=[
            pltpu.VMEM((KW, W), jnp.int32),
            pltpu.VMEM((KW, W), jnp.int32),
            pltpu.VMEM((W, D_H), jnp.float32),
            pltpu.VMEM((W, D_H), jnp.float32),
            pltpu.VMEM_SHARED((NP, D_H), jnp.float32),
            pltpu.SemaphoreType.DMA,
            pltpu.SemaphoreType.DMA,
        ],
    )
    def k(y_hbm, src_hbm, dst_hbm, z_hbm, out_hbm,
          src_v, dst_v, rows0, rows1, acc, sem0, sem1):
        cid = lax.axis_index("c")
        sid = lax.axis_index("s")
        base = sid * RPT
        pltpu.sync_copy(z_hbm.at[pl.ds(base, RPT)], acc.at[pl.ds(base, RPT)])
        pltpu.sync_copy(src_hbm.at[cid].at[sid], src_v)
        pltpu.sync_copy(dst_hbm.at[cid].at[sid], dst_v)
        plsc.subcore_barrier()

        pltpu.async_copy(y_hbm.at[src_v.at[0]], rows0, sem0)

        @pl.loop(0, KW, step=2)
        def _(t):
            pltpu.make_async_copy(y_hbm.at[src_v.at[t]], rows0, sem0).wait()
            pltpu.async_copy(y_hbm.at[src_v.at[t + 1]], rows1, sem1)
            pltpu.sync_copy(rows0, acc.at[dst_v.at[t]], add=True)
            pltpu.make_async_copy(y_hbm.at[src_v.at[t + 1]], rows1, sem1).wait()

            @pl.when(t + 2 < KW)
            def _():
                pltpu.async_copy(y_hbm.at[src_v.at[t + 2]], rows0, sem0)

            pltpu.sync_copy(rows1, acc.at[dst_v.at[t + 1]], add=True)

        plsc.subcore_barrier()
        pltpu.sync_copy(acc.at[pl.ds(base, RPT)],
                        out_hbm.at[cid].at[pl.ds(base, RPT)])

    return k(y, srcw, dstw, zeros64)


def _tc_layer1(cnt, x, W1):
    def body(c_ref, x_ref, w_ref, y_ref, d_ref):
        c = c_ref[0][:, 0:1] + c_ref[1][:, 0:1]
        d = lax.rsqrt(c + 1.0)[:N]
        u = jax.lax.dot_general(
            x_ref[...], w_ref[...], (((1,), (0,)), ((), ())),
            preferred_element_type=jnp.float32,
            precision=lax.Precision.HIGHEST)
        y_ref[...] = u * d
        d_ref[...] = d

    return pl.pallas_call(
        body,
        out_shape=(jax.ShapeDtypeStruct((N, D_H), jnp.float32),
                   jax.ShapeDtypeStruct((N, 1), jnp.float32)),
    )(cnt, x, W1)


def _tc_layer2(parts1, y1, d, Wcat, b1):
    def body(p_ref, y1_ref, d_ref, w_ref, b_ref, y2_ref):
        s1 = p_ref[0][:N] + p_ref[1][:N]
        dv = d_ref[...]
        h = jnp.maximum(dv * (s1 + y1_ref[...]) + b_ref[...], 0.0)
        y2_ref[...] = jax.lax.dot_general(
            h, w_ref[...], (((1,), (0,)), ((), ())),
            preferred_element_type=jnp.float32,
            precision=lax.Precision.HIGHEST) * dv

    return pl.pallas_call(
        body,
        out_shape=jax.ShapeDtypeStruct((N, D_H), jnp.float32),
    )(parts1, y1, d, Wcat, b1)


def _tc_out(parts2, y2, d, bcat):
    def body(p_ref, y2_ref, d_ref, b_ref, mu_ref, ls_ref):
        s2 = p_ref[0][:N] + p_ref[1][:N]
        o = d_ref[...] * (s2 + y2_ref[...]) + b_ref[...]
        mu_ref[...] = o[:, :D_OUT]
        ls_ref[...] = o[:, D_OUT:]

    return pl.pallas_call(
        body,
        out_shape=(jax.ShapeDtypeStruct((N, D_OUT), jnp.float32),
                   jax.ShapeDtypeStruct((N, D_OUT), jnp.float32)),
    )(parts2, y2, d, bcat)


def kernel(x, edge_index, W1, b1, Wmu, bmu, Wls, bls):
    src = edge_index[0]
    dst = edge_index[1]
    pad = EP - E
    srcw = jnp.concatenate(
        [src, jnp.zeros((pad,), jnp.int32)]).reshape(NC, NS, KW, W)
    pad_dst = N + (jnp.arange(pad, dtype=jnp.int32) % (NP - N))
    dstw = jnp.concatenate([dst, pad_dst]).reshape(NC, NS, KW, W)
    zeros1 = jnp.zeros((NP, 16), jnp.float32)
    zeros64 = jnp.zeros((NP, D_H), jnp.float32)
    ones1 = jnp.zeros((W, 16), jnp.float32).at[:, 0].set(1.0)

    cnt = _sc_degree(dstw, ones1, zeros1)
    y1, d = _tc_layer1(cnt, x, W1)
    parts1 = _sc_segment_sum(y1, srcw, dstw, zeros64)
    Wcat = jnp.concatenate([Wmu, Wls], axis=1)
    bcat = jnp.concatenate([bmu, bls]).reshape(1, 2 * D_OUT)
    y2 = _tc_layer2(parts1, y1, d, Wcat, b1.reshape(1, D_H))
    parts2 = _sc_segment_sum(y2, srcw, dstw, zeros64)
    mu, logstd = _tc_out(parts2, y2, d, bcat)
    return (mu, logstd)


# speedup vs baseline: 39.8868x; 1.9274x over previous
import functools

import jax
import jax.numpy as jnp
from jax import lax
from jax.experimental import pallas as pl
from jax.experimental.pallas import tpu as pltpu
from jax.experimental.pallas import tpu_sc as plsc

N = 10000
E = 320000
D_IN = 128
D_H = 64
D_OUT = 32

NC = 2
NS = 16
W = 128
KW = 80
EP = NC * NS * KW * W
NP = 10240
RPT = NP // NS

_MESH = plsc.VectorSubcoreMesh(core_axis_name="c", subcore_axis_name="s")
_SC_PARAMS = pltpu.CompilerParams(use_tc_tiling_on_sc=False)


def _sc_degree(dstw, ones1, zeros1):
    @functools.partial(
        pl.kernel,
        out_type=jax.ShapeDtypeStruct((NC, NP, 16), jnp.float32),
        mesh=_MESH,
        compiler_params=_SC_PARAMS,
        scratch_types=[
            pltpu.VMEM((KW, W), jnp.int32),
            pltpu.VMEM((W, 16), jnp.float32),
            pltpu.VMEM_SHARED((NP, 16), jnp.float32),
        ],
    )
    def k(dst_hbm, ones_hbm, z_hbm, out_hbm, dst_v, ones_v, acc):
        cid = lax.axis_index("c")
        sid = lax.axis_index("s")
        base = sid * RPT
        pltpu.sync_copy(z_hbm.at[pl.ds(base, RPT)], acc.at[pl.ds(base, RPT)])
        pltpu.sync_copy(ones_hbm, ones_v)
        pltpu.sync_copy(dst_hbm.at[cid].at[sid], dst_v)
        plsc.subcore_barrier()

        @pl.loop(0, KW)
        def _(t):
            pltpu.sync_copy(ones_v, acc.at[dst_v.at[t]], add=True)

        plsc.subcore_barrier()
        pltpu.sync_copy(acc.at[pl.ds(base, RPT)],
                        out_hbm.at[cid].at[pl.ds(base, RPT)])

    return k(dstw, ones1, zeros1)


def _sc_segment_sum(y, srcw, dstw, zeros64):
    @functools.partial(
        pl.kernel,
        out_type=jax.ShapeDtypeStruct((NC, NP, D_H), jnp.float32),
        mesh=_MESH,
        compiler_params=_SC_PARAMS,
        scratch_types=[
            pltpu.VMEM((KW, W), jnp.int32),
            pltpu.VMEM((KW, W), jnp.int32),
            pltpu.VMEM((W, D_H), jnp.float32),
            pltpu.VMEM((W, D_H), jnp.float32),
            pltpu.VMEM_SHARED((NP, D_H), jnp.float32),
            pltpu.VMEM_SHARED((N, D_H), jnp.float32),
            pltpu.SemaphoreType.DMA,
            pltpu.SemaphoreType.DMA,
        ],
    )
    def k(y_hbm, src_hbm, dst_hbm, z_hbm, out_hbm,
          src_v, dst_v, rows0, rows1, acc, y_sh, sem0, sem1):
        cid = lax.axis_index("c")
        sid = lax.axis_index("s")
        base = sid * RPT
        ybase = sid * (N // NS)
        pltpu.sync_copy(y_hbm.at[pl.ds(ybase, N // NS)],
                        y_sh.at[pl.ds(ybase, N // NS)])
        pltpu.sync_copy(z_hbm.at[pl.ds(base, RPT)], acc.at[pl.ds(base, RPT)])
        pltpu.sync_copy(src_hbm.at[cid].at[sid], src_v)
        pltpu.sync_copy(dst_hbm.at[cid].at[sid], dst_v)
        plsc.subcore_barrier()

        pltpu.async_copy(y_sh.at[src_v.at[0]], rows0, sem0)

        @pl.loop(0, KW, step=2)
        def _(t):
            pltpu.make_async_copy(y_sh.at[src_v.at[t]], rows0, sem0).wait()
            pltpu.async_copy(y_sh.at[src_v.at[t + 1]], rows1, sem1)
            pltpu.sync_copy(rows0, acc.at[dst_v.at[t]], add=True)
            pltpu.make_async_copy(y_sh.at[src_v.at[t + 1]], rows1, sem1).wait()

            @pl.when(t + 2 < KW)
            def _():
                pltpu.async_copy(y_sh.at[src_v.at[t + 2]], rows0, sem0)

            pltpu.sync_copy(rows1, acc.at[dst_v.at[t + 1]], add=True)

        plsc.subcore_barrier()
        pltpu.sync_copy(acc.at[pl.ds(base, RPT)],
                        out_hbm.at[cid].at[pl.ds(base, RPT)])

    return k(y, srcw, dstw, zeros64)


def _tc_layer1(cnt, x, W1):
    def body(c_ref, x_ref, w_ref, y_ref, d_ref):
        c = c_ref[0][:, 0:1] + c_ref[1][:, 0:1]
        d = lax.rsqrt(c + 1.0)[:N]
        u = jax.lax.dot_general(
            x_ref[...], w_ref[...], (((1,), (0,)), ((), ())),
            preferred_element_type=jnp.float32,
            precision=lax.Precision.HIGHEST)
        y_ref[...] = u * d
        d_ref[...] = d

    return pl.pallas_call(
        body,
        out_shape=(jax.ShapeDtypeStruct((N, D_H), jnp.float32),
                   jax.ShapeDtypeStruct((N, 1), jnp.float32)),
    )(cnt, x, W1)


def _tc_layer2(parts1, y1, d, Wcat, b1):
    def body(p_ref, y1_ref, d_ref, w_ref, b_ref, y2_ref):
        s1 = p_ref[0][:N] + p_ref[1][:N]
        dv = d_ref[...]
        h = jnp.maximum(dv * (s1 + y1_ref[...]) + b_ref[...], 0.0)
        y2_ref[...] = jax.lax.dot_general(
            h, w_ref[...], (((1,), (0,)), ((), ())),
            preferred_element_type=jnp.float32,
            precision=lax.Precision.HIGHEST) * dv

    return pl.pallas_call(
        body,
        out_shape=jax.ShapeDtypeStruct((N, D_H), jnp.float32),
    )(parts1, y1, d, Wcat, b1)


def _tc_out(parts2, y2, d, bcat):
    def body(p_ref, y2_ref, d_ref, b_ref, mu_ref, ls_ref):
        s2 = p_ref[0][:N] + p_ref[1][:N]
        o = d_ref[...] * (s2 + y2_ref[...]) + b_ref[...]
        mu_ref[...] = o[:, :D_OUT]
        ls_ref[...] = o[:, D_OUT:]

    return pl.pallas_call(
        body,
        out_shape=(jax.ShapeDtypeStruct((N, D_OUT), jnp.float32),
                   jax.ShapeDtypeStruct((N, D_OUT), jnp.float32)),
    )(parts2, y2, d, bcat)


def kernel(x, edge_index, W1, b1, Wmu, bmu, Wls, bls):
    src = edge_index[0]
    dst = edge_index[1]
    pad = EP - E
    srcw = jnp.concatenate(
        [src, jnp.zeros((pad,), jnp.int32)]).reshape(NC, NS, KW, W)
    pad_dst = N + (jnp.arange(pad, dtype=jnp.int32) % (NP - N))
    dstw = jnp.concatenate([dst, pad_dst]).reshape(NC, NS, KW, W)
    zeros1 = jnp.zeros((NP, 16), jnp.float32)
    zeros64 = jnp.zeros((NP, D_H), jnp.float32)
    ones1 = jnp.zeros((W, 16), jnp.float32).at[:, 0].set(1.0)

    cnt = _sc_degree(dstw, ones1, zeros1)
    y1, d = _tc_layer1(cnt, x, W1)
    parts1 = _sc_segment_sum(y1, srcw, dstw, zeros64)
    Wcat = jnp.concatenate([Wmu, Wls], axis=1)
    bcat = jnp.concatenate([bmu, bls]).reshape(1, 2 * D_OUT)
    y2 = _tc_layer2(parts1, y1, d, Wcat, b1.reshape(1, D_H))
    parts2 = _sc_segment_sum(y2, srcw, dstw, zeros64)
    mu, logstd = _tc_out(parts2, y2, d, bcat)
    return (mu, logstd)


# speedup vs baseline: 43.5346x; 1.0915x over previous
import functools

import jax
import jax.numpy as jnp
from jax import lax
from jax.experimental import pallas as pl
from jax.experimental.pallas import tpu as pltpu
from jax.experimental.pallas import tpu_sc as plsc

N = 10000
E = 320000
D_IN = 128
D_H = 64
D_OUT = 32

NC = 2
NS = 16
W = 128
KW = 80
EP = NC * NS * KW * W
NP = 10240
RPT = NP // NS

_MESH = plsc.VectorSubcoreMesh(core_axis_name="c", subcore_axis_name="s")
_SC_PARAMS = pltpu.CompilerParams(use_tc_tiling_on_sc=False)


def _sc_degree(dstw, ones1, zeros1):
    @functools.partial(
        pl.kernel,
        out_type=jax.ShapeDtypeStruct((NP, 128), jnp.float32),
        mesh=_MESH,
        compiler_params=_SC_PARAMS,
        scratch_types=[
            pltpu.VMEM((KW, W), jnp.int32),
            pltpu.VMEM((W, 16), jnp.float32),
            pltpu.VMEM_SHARED((NP, 16), jnp.float32),
        ],
    )
    def k(dst_hbm, ones_hbm, z_hbm, out_hbm, dst_v, ones_v, acc):
        cid = lax.axis_index("c")
        sid = lax.axis_index("s")
        base = sid * RPT
        pltpu.sync_copy(z_hbm.at[pl.ds(base, RPT)], acc.at[pl.ds(base, RPT)])
        pltpu.sync_copy(ones_hbm, ones_v)
        pltpu.sync_copy(dst_hbm.at[cid].at[sid], dst_v)
        plsc.subcore_barrier()

        @pl.loop(0, KW)
        def _(t):
            pltpu.sync_copy(ones_v, acc.at[dst_v.at[t]], add=True)

        plsc.subcore_barrier()
        pltpu.sync_copy(acc.at[pl.ds(base, RPT)],
                        out_hbm.at[pl.ds(base, RPT), pl.ds(cid * 16, 16)])

    return k(dstw, ones1, zeros1)


def _sc_segment_sum(y, srcw, dstw, zeros64):
    @functools.partial(
        pl.kernel,
        out_type=jax.ShapeDtypeStruct((NP, 128), jnp.float32),
        mesh=_MESH,
        compiler_params=_SC_PARAMS,
        scratch_types=[
            pltpu.VMEM((KW, W), jnp.int32),
            pltpu.VMEM((KW, W), jnp.int32),
            pltpu.VMEM((W, D_H), jnp.float32),
            pltpu.VMEM((W, D_H), jnp.float32),
            pltpu.VMEM_SHARED((NP, D_H), jnp.float32),
            pltpu.VMEM_SHARED((N, D_H), jnp.float32),
            pltpu.SemaphoreType.DMA,
            pltpu.SemaphoreType.DMA,
        ],
    )
    def k(y_hbm, src_hbm, dst_hbm, z_hbm, out_hbm,
          src_v, dst_v, rows0, rows1, acc, y_sh, sem0, sem1):
        cid = lax.axis_index("c")
        sid = lax.axis_index("s")
        base = sid * RPT
        ybase = sid * (N // NS)
        pltpu.sync_copy(y_hbm.at[pl.ds(ybase, N // NS)],
                        y_sh.at[pl.ds(ybase, N // NS)])
        pltpu.sync_copy(z_hbm.at[pl.ds(base, RPT)], acc.at[pl.ds(base, RPT)])
        pltpu.sync_copy(src_hbm.at[cid].at[sid], src_v)
        pltpu.sync_copy(dst_hbm.at[cid].at[sid], dst_v)
        plsc.subcore_barrier()

        pltpu.async_copy(y_sh.at[src_v.at[0]], rows0, sem0)

        @pl.loop(0, KW, step=2)
        def _(t):
            pltpu.make_async_copy(y_sh.at[src_v.at[t]], rows0, sem0).wait()
            pltpu.async_copy(y_sh.at[src_v.at[t + 1]], rows1, sem1)
            pltpu.sync_copy(rows0, acc.at[dst_v.at[t]], add=True)
            pltpu.make_async_copy(y_sh.at[src_v.at[t + 1]], rows1, sem1).wait()

            @pl.when(t + 2 < KW)
            def _():
                pltpu.async_copy(y_sh.at[src_v.at[t + 2]], rows0, sem0)

            pltpu.sync_copy(rows1, acc.at[dst_v.at[t + 1]], add=True)

        plsc.subcore_barrier()
        pltpu.sync_copy(acc.at[pl.ds(base, RPT)],
                        out_hbm.at[pl.ds(base, RPT), pl.ds(cid * D_H, D_H)])

    return k(y, srcw, dstw, zeros64)


def _tc_layer1(cnt, x, W1):
    def body(c_ref, x_ref, w_ref, y_ref, d_ref):
        c = c_ref[:, 0:1] + c_ref[:, 16:17]
        d = lax.rsqrt(c + 1.0)[:N]
        u = jax.lax.dot_general(
            x_ref[...], w_ref[...], (((1,), (0,)), ((), ())),
            preferred_element_type=jnp.float32,
            precision=lax.Precision.HIGHEST)
        y_ref[...] = u * d
        d_ref[...] = d

    return pl.pallas_call(
        body,
        out_shape=(jax.ShapeDtypeStruct((N, D_H), jnp.float32),
                   jax.ShapeDtypeStruct((N, 1), jnp.float32)),
    )(cnt, x, W1)


def _tc_layer2(parts1, y1, d, Wcat, b1):
    def body(p_ref, y1_ref, d_ref, w_ref, b_ref, y2_ref):
        s1 = p_ref[:N, :D_H] + p_ref[:N, D_H:]
        dv = d_ref[...]
        h = jnp.maximum(dv * (s1 + y1_ref[...]) + b_ref[...], 0.0)
        y2_ref[...] = jax.lax.dot_general(
            h, w_ref[...], (((1,), (0,)), ((), ())),
            preferred_element_type=jnp.float32,
            precision=lax.Precision.HIGHEST) * dv

    return pl.pallas_call(
        body,
        out_shape=jax.ShapeDtypeStruct((N, D_H), jnp.float32),
    )(parts1, y1, d, Wcat, b1)


def _tc_out(parts2, y2, d, bcat):
    def body(p_ref, y2_ref, d_ref, b_ref, mu_ref, ls_ref):
        s2 = p_ref[:N, :D_H] + p_ref[:N, D_H:]
        o = d_ref[...] * (s2 + y2_ref[...]) + b_ref[...]
        mu_ref[...] = o[:, :D_OUT]
        ls_ref[...] = o[:, D_OUT:]

    return pl.pallas_call(
        body,
        out_shape=(jax.ShapeDtypeStruct((N, D_OUT), jnp.float32),
                   jax.ShapeDtypeStruct((N, D_OUT), jnp.float32)),
    )(parts2, y2, d, bcat)


def kernel(x, edge_index, W1, b1, Wmu, bmu, Wls, bls):
    src = edge_index[0]
    dst = edge_index[1]
    pad = EP - E
    srcw = jnp.concatenate(
        [src, jnp.zeros((pad,), jnp.int32)]).reshape(NC, NS, KW, W)
    pad_dst = N + (jnp.arange(pad, dtype=jnp.int32) % (NP - N))
    dstw = jnp.concatenate([dst, pad_dst]).reshape(NC, NS, KW, W)
    zeros1 = jnp.zeros((NP, 16), jnp.float32)
    zeros64 = jnp.zeros((NP, D_H), jnp.float32)
    ones1 = jnp.zeros((W, 16), jnp.float32).at[:, 0].set(1.0)

    cnt = _sc_degree(dstw, ones1, zeros1)
    y1, d = _tc_layer1(cnt, x, W1)
    parts1 = _sc_segment_sum(y1, srcw, dstw, zeros64)
    Wcat = jnp.concatenate([Wmu, Wls], axis=1)
    bcat = jnp.concatenate([bmu, bls]).reshape(1, 2 * D_OUT)
    y2 = _tc_layer2(parts1, y1, d, Wcat, b1.reshape(1, D_H))
    parts2 = _sc_segment_sum(y2, srcw, dstw, zeros64)
    mu, logstd = _tc_out(parts2, y2, d, bcat)
    return (mu, logstd)
